# baseline (speedup 1.0000x reference)
import functools

import jax
import jax.numpy as jnp
from jax import lax
from jax.experimental import pallas as pl
from jax.experimental.pallas import tpu as pltpu
from jax.experimental.pallas import tpu_sc as plsc

NHEADS = 16
NREL = 3969
SIDE = 1024
LANES = 16

_info = plsc.get_sparse_core_info()
NCORES = _info.num_cores
NSUB = _info.num_subcores
NWORKERS = NCORES * NSUB

ROWS_PER_TILE = SIDE // NWORKERS
NBUF = 2


def _make_sc_gather():
    mesh = plsc.VectorSubcoreMesh(core_axis_name="c", subcore_axis_name="s")

    @functools.partial(
        pl.kernel,
        mesh=mesh,
        compiler_params=pltpu.CompilerParams(
            needs_layout_passes=False,
            disable_bounds_checks=True,
            disable_semaphore_checks=True,
            skip_device_barrier=True,
        ),
        out_type=jax.ShapeDtypeStruct((NHEADS, SIDE, SIDE), jnp.float32),
        scratch_types=[
            pltpu.VMEM((NHEADS, NREL), jnp.float32),
            pltpu.VMEM((NBUF, 1, SIDE), jnp.int32),
            pltpu.VMEM((NBUF, NHEADS, SIDE), jnp.float32),
            pltpu.SemaphoreType.DMA,
            pltpu.SemaphoreType.DMA,
            pltpu.SemaphoreType.DMA,
            pltpu.SemaphoreType.DMA,
        ],
    )
    def gather_kernel(table_hbm, idx_hbm, out_hbm, table_v, idx_v, out_v,
                      sem_idx0, sem_idx1, sem_out0, sem_out1):
        wid = lax.axis_index("s") * NCORES + lax.axis_index("c")
        row0 = wid * ROWS_PER_TILE
        sem_idx = (sem_idx0, sem_idx1)
        sem_out = (sem_out0, sem_out1)
        pltpu.sync_copy(table_hbm, table_v)

        pltpu.async_copy(idx_hbm.at[row0, :], idx_v.at[0, 0], sem_idx[0])

        def outer(i, carry):
            for b in range(NBUF):
                r = row0 + i * NBUF + b
                pltpu.make_async_copy(
                    idx_hbm.at[r, :], idx_v.at[b, 0], sem_idx[b]).wait()

                @pl.when(i * NBUF + b + 1 < ROWS_PER_TILE)
                def _prefetch():
                    pltpu.async_copy(
                        idx_hbm.at[r + 1, :], idx_v.at[1 - b, 0],
                        sem_idx[1 - b])

                @pl.when(i * NBUF + b >= NBUF)
                def _drain():
                    pltpu.make_async_copy(
                        out_v.at[b], out_hbm.at[:, r - NBUF, :],
                        sem_out[b]).wait()

                @plsc.parallel_loop(0, SIDE // LANES, unroll=4)
                def gather_body(j):
                    iv = idx_v[b, 0, pl.ds(j * LANES, LANES)]
                    for h in range(NHEADS):
                        hvec = jnp.full((LANES,), h, jnp.int32)
                        out_v[b, h, pl.ds(j * LANES, LANES)] = (
                            plsc.load_gather(table_v, [hvec, iv]))

                pltpu.async_copy(
                    out_v.at[b], out_hbm.at[:, r, :], sem_out[b])
            return carry

        lax.fori_loop(0, ROWS_PER_TILE // NBUF, outer, 0)

        for b in range(NBUF):
            r = row0 + ROWS_PER_TILE - NBUF + b
            pltpu.make_async_copy(
                out_v.at[b], out_hbm.at[:, r, :], sem_out[b]).wait()

    return gather_kernel


_sc_gather = _make_sc_gather()


def kernel(relative_bias_table, relative_position_index):
    idx = relative_position_index.astype(jnp.int32)
    return _sc_gather(relative_bias_table, idx)


# speedup vs baseline: 1.0736x; 1.0736x over previous
import functools

import jax
import jax.numpy as jnp
from jax import lax
from jax.experimental import pallas as pl
from jax.experimental.pallas import tpu as pltpu
from jax.experimental.pallas import tpu_sc as plsc

NHEADS = 16
NREL = 3969
SIDE = 1024
LANES = 16

_info = plsc.get_sparse_core_info()
NCORES = _info.num_cores
NSUB = _info.num_subcores
NWORKERS = NCORES * NSUB

ROWS_PER_TILE = SIDE // NWORKERS
NBUF = 3


def _make_sc_gather():
    mesh = plsc.VectorSubcoreMesh(core_axis_name="c", subcore_axis_name="s")

    @functools.partial(
        pl.kernel,
        mesh=mesh,
        compiler_params=pltpu.CompilerParams(
            needs_layout_passes=False,
            disable_bounds_checks=True,
            disable_semaphore_checks=True,
            skip_device_barrier=True,
        ),
        out_type=jax.ShapeDtypeStruct((NHEADS, SIDE, SIDE), jnp.float32),
        scratch_types=[
            pltpu.VMEM((NHEADS, NREL), jnp.float32),
            pltpu.VMEM((NBUF, 1, SIDE), jnp.int32),
            pltpu.VMEM((NBUF, NHEADS, SIDE), jnp.float32),
            pltpu.SemaphoreType.DMA,
            pltpu.SemaphoreType.DMA,
            pltpu.SemaphoreType.DMA,
            pltpu.SemaphoreType.DMA,
            pltpu.SemaphoreType.DMA,
            pltpu.SemaphoreType.DMA,
        ],
    )
    def gather_kernel(table_hbm, idx_hbm, out_hbm, table_v, idx_v, out_v,
                      sem_idx0, sem_idx1, sem_idx2,
                      sem_out0, sem_out1, sem_out2):
        wid = lax.axis_index("s") * NCORES + lax.axis_index("c")
        row0 = wid * ROWS_PER_TILE
        sem_idx = (sem_idx0, sem_idx1, sem_idx2)
        sem_out = (sem_out0, sem_out1, sem_out2)
        pltpu.sync_copy(table_hbm, table_v)

        pltpu.async_copy(idx_hbm.at[row0, :], idx_v.at[0, 0], sem_idx[0])
        pltpu.async_copy(idx_hbm.at[row0 + 1, :], idx_v.at[1, 0], sem_idx[1])

        def outer(i, carry):
            for b in range(NBUF):
                r = row0 + i * NBUF + b
                pltpu.make_async_copy(
                    idx_hbm.at[r, :], idx_v.at[b, 0], sem_idx[b]).wait()

                @pl.when(i * NBUF + b + 2 < ROWS_PER_TILE)
                def _prefetch():
                    nb = b + 2 if b + 2 < NBUF else b + 2 - NBUF
                    pltpu.async_copy(
                        idx_hbm.at[r + 2, :], idx_v.at[nb, 0], sem_idx[nb])

                @pl.when(i * NBUF + b >= NBUF)
                def _drain():
                    pltpu.make_async_copy(
                        out_v.at[b], out_hbm.at[:, r - NBUF, :],
                        sem_out[b]).wait()

                @plsc.parallel_loop(0, SIDE // LANES, unroll=4)
                def gather_body(j):
                    iv = idx_v[b, 0, pl.ds(j * LANES, LANES)]
                    for h in range(NHEADS):
                        hvec = jnp.full((LANES,), h, jnp.int32)
                        out_v[b, h, pl.ds(j * LANES, LANES)] = (
                            plsc.load_gather(table_v, [hvec, iv]))

                pltpu.async_copy(
                    out_v.at[b], out_hbm.at[:, r, :], sem_out[b])
            return carry

        lax.fori_loop(0, ROWS_PER_TILE // NBUF, outer, 0)

        for t in range(ROWS_PER_TILE % NBUF):
            b = t
            r = row0 + (ROWS_PER_TILE // NBUF) * NBUF + t
            pltpu.make_async_copy(
                idx_hbm.at[r, :], idx_v.at[b, 0], sem_idx[b]).wait()
            pltpu.make_async_copy(
                out_v.at[b], out_hbm.at[:, r - NBUF, :], sem_out[b]).wait()

            @plsc.parallel_loop(0, SIDE // LANES, unroll=4)
            def gather_body(j):
                iv = idx_v[b, 0, pl.ds(j * LANES, LANES)]
                for h in range(NHEADS):
                    hvec = jnp.full((LANES,), h, jnp.int32)
                    out_v[b, h, pl.ds(j * LANES, LANES)] = (
                        plsc.load_gather(table_v, [hvec, iv]))

            pltpu.async_copy(out_v.at[b], out_hbm.at[:, r, :], sem_out[b])

        for t in range(NBUF):
            r = row0 + ROWS_PER_TILE - NBUF + t
            b = (ROWS_PER_TILE - NBUF + t) % NBUF
            pltpu.make_async_copy(
                out_v.at[b], out_hbm.at[:, r, :], sem_out[b]).wait()

    return gather_kernel


_sc_gather = _make_sc_gather()


def kernel(relative_bias_table, relative_position_index):
    idx = relative_position_index.astype(jnp.int32)
    return _sc_gather(relative_bias_table, idx)


# speedup vs baseline: 1.1020x; 1.0264x over previous
import functools

import jax
import jax.numpy as jnp
from jax import lax
from jax.experimental import pallas as pl
from jax.experimental.pallas import tpu as pltpu
from jax.experimental.pallas import tpu_sc as plsc

NHEADS = 16
NREL = 3969
SIDE = 1024
LANES = 16

_info = plsc.get_sparse_core_info()
NCORES = _info.num_cores
NSUB = _info.num_subcores
NWORKERS = NCORES * NSUB

ROWS_PER_TILE = SIDE // NWORKERS
NBUF = 3


def _make_sc_gather():
    mesh = plsc.VectorSubcoreMesh(core_axis_name="c", subcore_axis_name="s")

    @functools.partial(
        pl.kernel,
        mesh=mesh,
        compiler_params=pltpu.CompilerParams(
            needs_layout_passes=False,
            disable_bounds_checks=True,
            disable_semaphore_checks=True,
            skip_device_barrier=True,
        ),
        out_type=jax.ShapeDtypeStruct((NHEADS, SIDE, SIDE), jnp.float32),
        scratch_types=[
            pltpu.VMEM((NHEADS, NREL), jnp.float32),
            pltpu.VMEM((NBUF, 1, SIDE), jnp.int32),
            pltpu.VMEM((NBUF, NHEADS, SIDE), jnp.float32),
            pltpu.SemaphoreType.DMA,
            pltpu.SemaphoreType.DMA,
            pltpu.SemaphoreType.DMA,
            pltpu.SemaphoreType.DMA,
            pltpu.SemaphoreType.DMA,
            pltpu.SemaphoreType.DMA,
        ],
    )
    def gather_kernel(table_hbm, idx_hbm, out_hbm, table_v, idx_v, out_v,
                      sem_idx0, sem_idx1, sem_idx2,
                      sem_out0, sem_out1, sem_out2):
        wid = lax.axis_index("s") * NCORES + lax.axis_index("c")
        row0 = wid * ROWS_PER_TILE
        sem_idx = (sem_idx0, sem_idx1, sem_idx2)
        sem_out = (sem_out0, sem_out1, sem_out2)
        pltpu.sync_copy(table_hbm, table_v)

        pltpu.async_copy(idx_hbm.at[row0, :], idx_v.at[0, 0], sem_idx[0])
        pltpu.async_copy(idx_hbm.at[row0 + 1, :], idx_v.at[1, 0], sem_idx[1])

        def outer(i, carry):
            for b in range(NBUF):
                r = row0 + i * NBUF + b
                pltpu.make_async_copy(
                    idx_hbm.at[r, :], idx_v.at[b, 0], sem_idx[b]).wait()

                @pl.when(i * NBUF + b + 2 < ROWS_PER_TILE)
                def _prefetch():
                    nb = b + 2 if b + 2 < NBUF else b + 2 - NBUF
                    pltpu.async_copy(
                        idx_hbm.at[r + 2, :], idx_v.at[nb, 0], sem_idx[nb])

                @pl.when(i * NBUF + b >= NBUF)
                def _drain():
                    pltpu.make_async_copy(
                        out_v.at[b], out_hbm.at[:, r - NBUF, :],
                        sem_out[b]).wait()

                @plsc.parallel_loop(0, SIDE // LANES, unroll=2)
                def gather_body(j):
                    iv = idx_v[b, 0, pl.ds(j * LANES, LANES)]
                    for h in range(NHEADS):
                        hvec = jnp.full((LANES,), h, jnp.int32)
                        out_v[b, h, pl.ds(j * LANES, LANES)] = (
                            plsc.load_gather(table_v, [hvec, iv]))

                pltpu.async_copy(
                    out_v.at[b], out_hbm.at[:, r, :], sem_out[b])
            return carry

        lax.fori_loop(0, ROWS_PER_TILE // NBUF, outer, 0)

        for t in range(ROWS_PER_TILE % NBUF):
            b = t
            r = row0 + (ROWS_PER_TILE // NBUF) * NBUF + t
            pltpu.make_async_copy(
                idx_hbm.at[r, :], idx_v.at[b, 0], sem_idx[b]).wait()
            pltpu.make_async_copy(
                out_v.at[b], out_hbm.at[:, r - NBUF, :], sem_out[b]).wait()

            @plsc.parallel_loop(0, SIDE // LANES, unroll=2)
            def gather_body(j):
                iv = idx_v[b, 0, pl.ds(j * LANES, LANES)]
                for h in range(NHEADS):
                    hvec = jnp.full((LANES,), h, jnp.int32)
                    out_v[b, h, pl.ds(j * LANES, LANES)] = (
                        plsc.load_gather(table_v, [hvec, iv]))

            pltpu.async_copy(out_v.at[b], out_hbm.at[:, r, :], sem_out[b])

        for t in range(NBUF):
            r = row0 + ROWS_PER_TILE - NBUF + t
            b = (ROWS_PER_TILE - NBUF + t) % NBUF
            pltpu.make_async_copy(
                out_v.at[b], out_hbm.at[:, r, :], sem_out[b]).wait()

    return gather_kernel


_sc_gather = _make_sc_gather()


def kernel(relative_bias_table, relative_position_index):
    idx = relative_position_index.astype(jnp.int32)
    return _sc_gather(relative_bias_table, idx)


# speedup vs baseline: 1.1026x; 1.0005x over previous
import functools

import jax
import jax.numpy as jnp
from jax import lax
from jax.experimental import pallas as pl
from jax.experimental.pallas import tpu as pltpu
from jax.experimental.pallas import tpu_sc as plsc

NHEADS = 16
NREL = 3969
SIDE = 1024
LANES = 16

_info = plsc.get_sparse_core_info()
NCORES = _info.num_cores
NSUB = _info.num_subcores
NWORKERS = NCORES * NSUB

ROWS_PER_TILE = SIDE // NWORKERS
NBUF = 3


def _make_sc_gather():
    mesh = plsc.VectorSubcoreMesh(core_axis_name="c", subcore_axis_name="s")

    @functools.partial(
        pl.kernel,
        mesh=mesh,
        compiler_params=pltpu.CompilerParams(
            needs_layout_passes=False,
            disable_bounds_checks=True,
            disable_semaphore_checks=True,
            skip_device_barrier=True,
        ),
        out_type=jax.ShapeDtypeStruct((NHEADS, SIDE, SIDE), jnp.float32),
        scratch_types=[
            pltpu.VMEM((NHEADS, NREL), jnp.float32),
            pltpu.VMEM((NBUF, 1, SIDE), jnp.int32),
            pltpu.VMEM((NBUF, NHEADS, SIDE), jnp.float32),
            pltpu.SemaphoreType.DMA,
            pltpu.SemaphoreType.DMA,
            pltpu.SemaphoreType.DMA,
            pltpu.SemaphoreType.DMA,
            pltpu.SemaphoreType.DMA,
            pltpu.SemaphoreType.DMA,
        ],
    )
    def gather_kernel(table_hbm, idx_hbm, out_hbm, table_v, idx_v, out_v,
                      sem_idx0, sem_idx1, sem_idx2,
                      sem_out0, sem_out1, sem_out2):
        wid = lax.axis_index("s") * NCORES + lax.axis_index("c")
        row0 = wid * ROWS_PER_TILE
        sem_idx = (sem_idx0, sem_idx1, sem_idx2)
        sem_out = (sem_out0, sem_out1, sem_out2)
        pltpu.sync_copy(table_hbm, table_v)

        pltpu.async_copy(idx_hbm.at[row0, :], idx_v.at[0, 0], sem_idx[0])
        pltpu.async_copy(idx_hbm.at[row0 + 1, :], idx_v.at[1, 0], sem_idx[1])

        def outer(i, carry):
            for b in range(NBUF):
                r = row0 + i * NBUF + b
                pltpu.make_async_copy(
                    idx_hbm.at[r, :], idx_v.at[b, 0], sem_idx[b]).wait()

                @pl.when(i * NBUF + b + 2 < ROWS_PER_TILE)
                def _prefetch():
                    nb = b + 2 if b + 2 < NBUF else b + 2 - NBUF
                    pltpu.async_copy(
                        idx_hbm.at[r + 2, :], idx_v.at[nb, 0], sem_idx[nb])

                @pl.when(i * NBUF + b >= NBUF)
                def _drain():
                    pltpu.make_async_copy(
                        out_v.at[b], out_hbm.at[:, r - NBUF, :],
                        sem_out[b]).wait()

                @plsc.parallel_loop(0, SIDE // LANES, unroll=1)
                def gather_body(j):
                    iv = idx_v[b, 0, pl.ds(j * LANES, LANES)]
                    for h in range(NHEADS):
                        hvec = jnp.full((LANES,), h, jnp.int32)
                        out_v[b, h, pl.ds(j * LANES, LANES)] = (
                            plsc.load_gather(table_v, [hvec, iv]))

                pltpu.async_copy(
                    out_v.at[b], out_hbm.at[:, r, :], sem_out[b])
            return carry

        lax.fori_loop(0, ROWS_PER_TILE // NBUF, outer, 0)

        for t in range(ROWS_PER_TILE % NBUF):
            b = t
            r = row0 + (ROWS_PER_TILE // NBUF) * NBUF + t
            pltpu.make_async_copy(
                idx_hbm.at[r, :], idx_v.at[b, 0], sem_idx[b]).wait()
            pltpu.make_async_copy(
                out_v.at[b], out_hbm.at[:, r - NBUF, :], sem_out[b]).wait()

            @plsc.parallel_loop(0, SIDE // LANES, unroll=1)
            def gather_body(j):
                iv = idx_v[b, 0, pl.ds(j * LANES, LANES)]
                for h in range(NHEADS):
                    hvec = jnp.full((LANES,), h, jnp.int32)
                    out_v[b, h, pl.ds(j * LANES, LANES)] = (
                        plsc.load_gather(table_v, [hvec, iv]))

            pltpu.async_copy(out_v.at[b], out_hbm.at[:, r, :], sem_out[b])

        for t in range(NBUF):
            r = row0 + ROWS_PER_TILE - NBUF + t
            b = (ROWS_PER_TILE - NBUF + t) % NBUF
            pltpu.make_async_copy(
                out_v.at[b], out_hbm.at[:, r, :], sem_out[b]).wait()

    return gather_kernel


_sc_gather = _make_sc_gather()


def kernel(relative_bias_table, relative_position_index):
    idx = relative_position_index.astype(jnp.int32)
    return _sc_gather(relative_bias_table, idx)
